# baseline (speedup 1.0000x reference)
from functools import partial

import jax
import jax.numpy as jnp
from jax import lax
from jax.experimental import pallas as pl
from jax.experimental.pallas import tpu as pltpu
from jax.experimental.pallas import tpu_sc as plsc

DMIN = 0.5
DMAX = 2.0
MITR = 12
HIDDEN = 16

_NT = 16
_L = 16

TOL_A = 3e-6
TOL_B = 1e-4
MAXIT_A = 150
MAXIT_B = 100


def _solve_body(N, NPT, EPT, XPAD, CH, tol, maxiter, node_rhs,
                pack_hbm, wneg_hbm, rhs_hbm, pws_hbm,
                fdir_hbm, pex_hbm, dots_hbm,
                pack_v, wneg_v, xf_v, xown_v, r_v, p_v, acc_v, dw_v,
                rhs_v, zbuf_v, dread_v, dwrite_v, aw_v):
    wid = lax.axis_index("s")
    base = wid * NPT
    NG = NPT // _L
    EG = EPT // _L
    NCH = EPT // CH
    CG_ = CH // _L
    APT = NPT + _L
    zeros = jnp.zeros((_L,), jnp.float32)
    nown = jnp.minimum(N - base, NPT)
    iota = lax.iota(jnp.int32, _L)
    plane_off = iota * APT

    pltpu.sync_copy(pack_hbm.at[wid], pack_v)
    pltpu.sync_copy(wneg_hbm.at[wid], wneg_v)
    xf_v[pl.ds(XPAD - _L, _L)] = zeros

    def aw_zero():
        def zb(i, carry):
            aw_v[pl.ds(i * _L, _L)] = zeros
            return carry

        lax.fori_loop(0, APT, zb, 0)

    def aw_reduce_into(dst_ref):
        def rb(i, carry):
            sl = pl.ds(i * _L, _L)
            tot = aw_v[sl]
            for l in range(1, _L):
                tot = tot + aw_v[pl.ds(l * APT + i * _L, _L)]
            dst_ref[sl] = dst_ref[sl] + tot
            return carry

        lax.fori_loop(0, NG, rb, 0)

    aw_zero()

    def deg_body(g, carry):
        sl = pl.ds(g * _L, _L)
        pk = pack_v[sl]
        wn = wneg_v[sl]
        rw = (pk >> 14) & 0x3FF
        plsc.addupdate_scatter(aw_v, [rw + plane_off], zeros - wn)
        return carry

    lax.fori_loop(0, EG, deg_body, 0)
    for k in range(NG):
        dw_v[pl.ds(k * _L, _L)] = zeros
    aw_reduce_into(dw_v)

    if node_rhs:
        pltpu.sync_copy(rhs_hbm.at[pl.ds(base, NPT)], rhs_v)
    else:
        aw_zero()

        def chunk_body(c, carry):
            pltpu.sync_copy(rhs_hbm.at[pl.ds(wid * EPT + c * CH, CH)], zbuf_v)

            def g_body(g, carry2):
                pk = pack_v[pl.ds(c * CH + g * _L, _L)]
                zv = zbuf_v[pl.ds(g * _L, _L)]
                rw = (pk >> 14) & 0x3FF
                sgn = (pk >> 24) & 1
                plsc.addupdate_scatter(aw_v, [rw + plane_off],
                                       jnp.where(sgn == 1, -zv, zv))
                return carry2

            lax.fori_loop(0, CG_, g_body, 0)
            return carry

        lax.fori_loop(0, NCH, chunk_body, 0)
        for k in range(NG):
            rhs_v[pl.ds(k * _L, _L)] = zeros
        aw_reduce_into(rhs_v)

    pltpu.sync_copy(pws_hbm.at[pl.ds(base, NPT)], xown_v)
    pltpu.sync_copy(pws_hbm, xf_v.at[pl.ds(0, NPT * _NT)])
    l0 = zeros
    l1 = zeros
    l2 = zeros
    for k in range(NG):
        sl = pl.ds(k * _L, _L)
        rv = rhs_v[sl]
        l0 = l0 + rv
        l1 = l1 + rv * rv
        l2 = l2 + xown_v[sl]
    dwrite_v[pl.ds(0, _L)] = l0
    pltpu.sync_copy(dwrite_v, dots_hbm.at[pl.ds(wid * _L, _L)])
    dwrite_v[pl.ds(0, _L)] = l1
    pltpu.sync_copy(dwrite_v, dots_hbm.at[pl.ds(256 + wid * _L, _L)])
    dwrite_v[pl.ds(0, _L)] = l2
    pltpu.sync_copy(dwrite_v, dots_hbm.at[pl.ds(512 + wid * _L, _L)])
    plsc.subcore_barrier()

    def read_slot(slot):
        pltpu.sync_copy(dots_hbm.at[pl.ds(slot * 256, 256)], dread_v)
        tot = zeros
        for t in range(_NT):
            tot = tot + dread_v[pl.ds(t * _L, _L)]
        return jnp.sum(tot)

    rsum = read_slot(0)
    rsq = read_slot(1)
    psum0 = read_slot(2)
    rmean = rsum * (1.0 / float(N))
    bnorm2 = rsq - float(N) * rmean * rmean
    thresh2 = (tol * tol) * bnorm2

    for k in range(NG):
        sl = pl.ds(k * _L, _L)
        msk = (iota + (k * _L)) < nown
        rhs_v[sl] = rhs_v[sl] - jnp.where(msk, rmean, 0.0)

    def matvec(psum, xs_ref):
        aw_zero()

        def e_body(g, carry):
            sl = pl.ds(g * _L, _L)
            pk = pack_v[sl]
            wn = wneg_v[sl]
            nb = pk & 0x3FFF
            rw = (pk >> 14) & 0x3FF
            xv = plsc.load_gather(xf_v, [nb])
            plsc.addupdate_scatter(aw_v, [rw + plane_off], wn * xv)
            return carry

        lax.fori_loop(0, EG, e_body, 0)
        smean = psum * (1.0 / float(N))
        for k in range(NG):
            sl = pl.ds(k * _L, _L)
            msk = (iota + (k * _L)) < nown
            acc_v[sl] = dw_v[sl] * xs_ref[sl] + jnp.where(msk, smean, 0.0)
        aw_reduce_into(acc_v)

    matvec(psum0, xown_v)
    lr = zeros
    lp = zeros
    for k in range(NG):
        sl = pl.ds(k * _L, _L)
        rv = rhs_v[sl] - acc_v[sl]
        r_v[sl] = rv
        p_v[sl] = rv
        lr = lr + rv * rv
        lp = lp + rv
    dwrite_v[pl.ds(0, _L)] = lr
    pltpu.sync_copy(dwrite_v, dots_hbm.at[pl.ds(wid * _L, _L)])
    dwrite_v[pl.ds(0, _L)] = lp
    pltpu.sync_copy(dwrite_v, dots_hbm.at[pl.ds(256 + wid * _L, _L)])
    pltpu.sync_copy(p_v, pex_hbm.at[pl.ds(base, NPT)])
    plsc.subcore_barrier()
    gamma0 = read_slot(0)
    psum = read_slot(1)
    pltpu.sync_copy(pex_hbm, xf_v.at[pl.ds(0, NPT * _NT)])

    def cond(st):
        k, gamma, _ = st
        return (k < maxiter) & (gamma > thresh2)

    def body(st):
        k, gamma, psum_ = st
        matvec(psum_, p_v)
        lpap = zeros
        for kk in range(NG):
            sl = pl.ds(kk * _L, _L)
            lpap = lpap + p_v[sl] * acc_v[sl]
        dwrite_v[pl.ds(0, _L)] = lpap
        pltpu.sync_copy(dwrite_v, dots_hbm.at[pl.ds(wid * _L, _L)])
        plsc.subcore_barrier()
        pap = read_slot(0)
        alpha_v = (zeros + gamma) / (zeros + pap)
        lrr = zeros
        for kk in range(NG):
            sl = pl.ds(kk * _L, _L)
            xv = xown_v[sl] + alpha_v * p_v[sl]
            rv = r_v[sl] - alpha_v * acc_v[sl]
            xown_v[sl] = xv
            r_v[sl] = rv
            lrr = lrr + rv * rv
        dwrite_v[pl.ds(0, _L)] = lrr
        pltpu.sync_copy(dwrite_v, dots_hbm.at[pl.ds(256 + wid * _L, _L)])
        plsc.subcore_barrier()
        g2 = read_slot(1)
        beta_v = (zeros + g2) / (zeros + gamma)
        lps = zeros
        for kk in range(NG):
            sl = pl.ds(kk * _L, _L)
            pv = r_v[sl] + beta_v * p_v[sl]
            p_v[sl] = pv
            lps = lps + pv
        pltpu.sync_copy(p_v, pex_hbm.at[pl.ds(base, NPT)])
        dwrite_v[pl.ds(0, _L)] = lps
        pltpu.sync_copy(dwrite_v, dots_hbm.at[pl.ds(512 + wid * _L, _L)])
        plsc.subcore_barrier()
        psum_n = read_slot(2)
        pltpu.sync_copy(pex_hbm, xf_v.at[pl.ds(0, NPT * _NT)])
        return k + 1, g2, psum_n

    lax.while_loop(cond, body, (jnp.int32(0), gamma0, psum))

    pltpu.sync_copy(xown_v, pex_hbm.at[pl.ds(base, NPT)])
    plsc.subcore_barrier()
    pltpu.sync_copy(pex_hbm, xf_v.at[pl.ds(0, NPT * _NT)])

    def ep_chunk(c, carry):
        if not node_rhs:
            pltpu.sync_copy(rhs_hbm.at[pl.ds(wid * EPT + c * CH, CH)], zbuf_v)

        def g_body(g, carry2):
            sl = pl.ds(g * _L, _L)
            pk = pack_v[pl.ds(c * CH + g * _L, _L)]
            nb = pk & 0x3FFF
            rw = (pk >> 14) & 0x3FF
            sgn = (pk >> 24) & 1
            pown = plsc.load_gather(xf_v, [rw + base])
            pnbr = plsc.load_gather(xf_v, [nb])
            diff = jnp.where(sgn == 1, pnbr - pown, pown - pnbr)
            if node_rhs:
                val = diff
            else:
                wn = wneg_v[pl.ds(c * CH + g * _L, _L)]
                val = zbuf_v[sl] + wn * diff
            zbuf_v[sl] = val
            return carry2

        lax.fori_loop(0, CG_, g_body, 0)
        pltpu.sync_copy(zbuf_v, fdir_hbm.at[pl.ds(wid * EPT + c * CH, CH)])
        return carry

    lax.fori_loop(0, NCH, ep_chunk, 0)


def _make_solve(N, N_PAD, NPT, EPT, XPAD, CH, tol, maxiter, node_rhs):
    NCH = EPT // CH
    mesh = plsc.VectorSubcoreMesh(core_axis_name="c", subcore_axis_name="s",
                                  num_cores=1)
    return pl.kernel(
        partial(_solve_body, N, NPT, EPT, XPAD, CH, tol, maxiter, node_rhs),
        out_type=(
            jax.ShapeDtypeStruct((_NT * EPT,), jnp.float32),
            jax.ShapeDtypeStruct((N_PAD,), jnp.float32),
            jax.ShapeDtypeStruct((768,), jnp.float32),
        ),
        mesh=mesh,
        compiler_params=pltpu.CompilerParams(needs_layout_passes=False),
        scratch_types=[
            pltpu.VMEM((EPT,), jnp.int32),
            pltpu.VMEM((EPT,), jnp.float32),
            pltpu.VMEM((XPAD,), jnp.float32),
            pltpu.VMEM((NPT,), jnp.float32),
            pltpu.VMEM((NPT,), jnp.float32),
            pltpu.VMEM((NPT,), jnp.float32),
            pltpu.VMEM((NPT + _L,), jnp.float32),
            pltpu.VMEM((NPT,), jnp.float32),
            pltpu.VMEM((NPT,), jnp.float32),
            pltpu.VMEM((CH,), jnp.float32),
            pltpu.VMEM((256,), jnp.float32),
            pltpu.VMEM((_L,), jnp.float32),
            pltpu.VMEM(((NPT + _L) * _L,), jnp.float32),
        ],
    )


def _z_body(scal_ref, fcy_ref, fcut_ref, w_ref, z_ref):
    fc = fcy_ref[...]
    wv = w_ref[...]
    y = (fc + fcut_ref[...]) / wv
    acc = (1.0 / DMAX) * y
    for k in range(HIDDEN):
        t = y * scal_ref[1, k] + scal_ref[2, k]
        sp = jnp.log1p(jnp.exp(-jnp.abs(t))) + jnp.maximum(t, 0.0)
        acc = acc + scal_ref[0, k] * sp
    z_ref[...] = fc - DMIN * wv * acc


def _make_zmap(rows):
    blk = 8
    for cand in range(8, min(rows, 2048) + 1, 8):
        if rows % cand == 0:
            blk = cand
    grid = rows // blk
    return pl.pallas_call(
        _z_body,
        grid=(grid,),
        in_specs=[
            pl.BlockSpec((8, 128), lambda i: (0, 0)),
            pl.BlockSpec((blk, 128), lambda i: (i, 0)),
            pl.BlockSpec((blk, 128), lambda i: (i, 0)),
            pl.BlockSpec((blk, 128), lambda i: (i, 0)),
        ],
        out_specs=pl.BlockSpec((blk, 128), lambda i: (i, 0)),
        out_shape=jax.ShapeDtypeStruct((rows, 128), jnp.float32),
    )


def _invscat_body(EPT, EPB, SENT, pos_hbm, out_hbm, pos_v, val_v, fill_v):
    wid = lax.axis_index("s")
    CPT = pos_v.shape[0]
    FCH = fill_v.shape[0]
    iota = lax.iota(jnp.int32, _L)
    sent = jnp.zeros((_L,), jnp.int32) + SENT

    def fb(i, carry):
        fill_v[pl.ds(i * _L, _L)] = sent
        return carry

    lax.fori_loop(0, FCH // _L, fb, 0)
    for c in range(EPT // FCH):
        pltpu.sync_copy(fill_v, out_hbm.at[pl.ds(wid * EPT + c * FCH, FCH)])

    @pl.when(wid == 0)
    def _():
        pltpu.sync_copy(fill_v.at[pl.ds(0, 128)],
                        out_hbm.at[pl.ds(_NT * EPT, 128)])

    pltpu.sync_copy(pos_hbm.at[wid], pos_v)
    plsc.subcore_barrier()

    def cb(c, carry):
        base = wid * EPB + c * 128
        for g in range(128 // _L):
            val_v[pl.ds(g * _L, _L)] = base + (g * _L) + iota
        pltpu.sync_copy(val_v, out_hbm.at[pos_v.at[c]])
        return carry

    lax.fori_loop(0, CPT, cb, 0)


def _make_invscat(EPT, TPC, EPB, SENT):
    CPT = TPC // 128
    mesh = plsc.VectorSubcoreMesh(core_axis_name="c", subcore_axis_name="s",
                                  num_cores=1)
    return pl.kernel(
        partial(_invscat_body, EPT, EPB, SENT),
        out_type=jax.ShapeDtypeStruct((_NT * EPT + 128,), jnp.int32),
        mesh=mesh,
        compiler_params=pltpu.CompilerParams(needs_layout_passes=False),
        scratch_types=[
            pltpu.VMEM((CPT, 128), jnp.int32),
            pltpu.VMEM((128,), jnp.int32),
            pltpu.VMEM((EPT // 8, ), jnp.int32),
        ],
    )


def _toedge_body(NROW, fcy_hbm, fcut_hbm, eid_hbm, out_hbm,
                 eid_v, b1_v, b2_v, sem):
    wid = lax.axis_index("s")
    EPT = NROW * 128
    pltpu.sync_copy(eid_hbm.at[wid], eid_v)

    def blk_body(c, carry):
        pltpu.sync_copy(fcy_hbm.at[pl.ds(wid * EPT + c * 1024, 1024)], b1_v)
        pltpu.sync_copy(fcut_hbm.at[pl.ds(wid * EPT + c * 1024, 1024)], b2_v)
        for g in range(1024 // _L):
            sl = pl.ds(g * _L, _L)
            b1_v[sl] = b1_v[sl] + b2_v[sl]
        descs = [pltpu.async_copy(b1_v.at[pl.ds(j * 128, 128)],
                                  out_hbm.at[eid_v.at[c * 8 + j]], sem)
                 for j in range(8)]
        for d in descs:
            d.wait()
        return carry

    lax.fori_loop(0, NROW // 8, blk_body, 0)


def _make_toedge(NROW, M_PAD):
    mesh = plsc.VectorSubcoreMesh(core_axis_name="c", subcore_axis_name="s",
                                  num_cores=1)
    return pl.kernel(
        partial(_toedge_body, NROW),
        out_type=jax.ShapeDtypeStruct((M_PAD,), jnp.float32),
        mesh=mesh,
        compiler_params=pltpu.CompilerParams(needs_layout_passes=False),
        scratch_types=[
            pltpu.VMEM((NROW, 128), jnp.int32),
            pltpu.VMEM((1024,), jnp.float32),
            pltpu.VMEM((1024,), jnp.float32),
            pltpu.SemaphoreType.DMA,
        ],
    )


def kernel(u, edge_index, edge_weights, alpha, log_s, b, num_nodes):
    N = u.shape[0]
    M = edge_weights.shape[0]
    M2 = 2 * M
    NPT = _L * ((N + _NT * _L - 1) // (_NT * _L))
    N_PAD = _NT * NPT
    XPAD = N_PAD + _L
    EPT = 1024 * ((M2 // _NT + 4200 + 1023) // 1024)
    NROW = EPT // 128
    CH = EPT // 10 if EPT % 10 == 0 and (EPT // 10) % _L == 0 else EPT // 8
    while EPT % CH or CH % _L:
        CH -= _L
    NCH = EPT // CH

    src = edge_index[0].astype(jnp.int32)
    dst = edge_index[1].astype(jnp.int32)
    w = edge_weights

    nodes = jnp.concatenate([src, dst])
    nbrs = jnp.concatenate([dst, src])
    wd = jnp.concatenate([w, w])
    sg = jnp.concatenate([jnp.zeros((M,), jnp.int32),
                          jnp.ones((M,), jnp.int32)])
    eid = jnp.concatenate([jnp.arange(M, dtype=jnp.int32)] * 2)
    tile = nodes // NPT
    row = nodes - tile * NPT
    K = 512
    M2P = K * ((M2 + K - 1) // K)
    tile_p = jnp.concatenate(
        [tile, jnp.full((M2P - M2,), _NT + 1, jnp.int32)]) if M2P > M2 else tile
    ohf = (tile_p[:, None] == jnp.arange(_NT, dtype=jnp.int32)[None, :]
           ).astype(jnp.float32).reshape(M2P // K, K, _NT)
    bsum = ohf.sum(axis=1)
    basec = jnp.cumsum(bsum, axis=0) - bsum
    tri = jnp.tril(jnp.ones((K, K), jnp.float32))
    within = jnp.einsum("kj,bjt->bkt", tri, ohf,
                        preferred_element_type=jnp.float32)
    rank_all = (basec[:, None, :] + within).reshape(M2P, _NT)[:M2]
    rank = jnp.take_along_axis(
        rank_all, tile[:, None], axis=1)[:, 0].astype(jnp.int32) - 1
    valid = rank < EPT
    pack_vals = nbrs | (row << 14) | (sg << 24)
    DUMMY = _NT * EPT
    arang = jnp.arange(M2, dtype=jnp.int32)
    pos = jnp.where(valid, tile * EPT + rank, DUMMY + (arang % 128))
    EPB = M2 // _NT
    TPC = 128 * ((EPB + 127) // 128)
    pos3 = jnp.full((_NT, TPC - EPB), DUMMY, jnp.int32)
    pos3 = jnp.concatenate([pos.reshape(_NT, EPB), pos3],
                           axis=1).reshape(_NT, TPC // 128, 128)
    invscat = _make_invscat(EPT, TPC, EPB, M2)
    inv = invscat(pos3)[:_NT * EPT]
    filled = inv < M2
    invc = jnp.minimum(inv, M2 - 1)
    lane = jnp.arange(_NT * EPT, dtype=jnp.int32) % _L
    pack = jnp.where(filled, pack_vals[invc],
                     (NPT + lane) << 14).reshape(_NT, EPT)
    wneg = jnp.where(filled, -wd[invc], 0.0).reshape(_NT, EPT)
    uneg = jnp.where(filled, -1.0, 0.0).reshape(_NT, EPT)
    wtc = jnp.where(filled, wd[invc], 1.0).reshape(_NT * EPT // 128, 128)
    eidp = jnp.where(filled, eid[invc], M).reshape(_NT, NROW, 128)

    solve_a = _make_solve(N, N_PAD, NPT, EPT, XPAD, CH, TOL_A, MAXIT_A, True)
    solve_b = _make_solve(N, N_PAD, NPT, EPT, XPAD, CH, TOL_B, MAXIT_B, False)
    zmap = _make_zmap(_NT * EPT // 128)
    toedge = _make_toedge(NROW, M + 128)

    u_pad = jnp.zeros((N_PAD,), jnp.float32).at[:N].set(u)
    zeros_p = jnp.zeros((N_PAD,), jnp.float32)

    fcut_flat, _, _ = solve_a(pack, uneg, u_pad, zeros_p)
    fcut_rows = fcut_flat.reshape(_NT * EPT // 128, 128)

    pvec = jax.nn.softmax(alpha)
    svec = jnp.exp(log_s)
    scal = jnp.zeros((8, 128), jnp.float32)
    scal = scal.at[0, :HIDDEN].set(pvec * (1.0 / DMIN - 1.0 / DMAX) / svec)
    scal = scal.at[1, :HIDDEN].set(svec)
    scal = scal.at[2, :HIDDEN].set(b)

    def body(i, carry):
        f_cyc_rows, p_ws = carry
        z_rows = zmap(scal, f_cyc_rows, fcut_rows, wtc)
        f_new_flat, p_new, _ = solve_b(pack, wneg,
                                       z_rows.reshape(-1), p_ws)
        return f_new_flat.reshape(_NT * EPT // 128, 128), p_new

    f0 = jnp.zeros((_NT * EPT // 128, 128), jnp.float32)
    f_cyc_rows, _ = lax.fori_loop(0, MITR, body, (f0, zeros_p))

    f_pad = toedge(f_cyc_rows.reshape(-1), fcut_flat, eidp)
    return f_pad[:M]


# speedup vs baseline: 2.9797x; 2.9797x over previous
from functools import partial

import jax
import jax.numpy as jnp
from jax import lax
from jax.experimental import pallas as pl
from jax.experimental.pallas import tpu as pltpu
from jax.experimental.pallas import tpu_sc as plsc

DMIN = 0.5
DMAX = 2.0
MITR = 12
HIDDEN = 16

_NT = 16
_L = 16

TOL_A = 3e-6
TOL_B = 1e-4
MAXIT_A = 150
MAXIT_B = 100


def _solve_body(N, NPT, EPT, XPAD, CH, tol, maxiter, node_rhs,
                pack_hbm, wneg_hbm, rhs_hbm, pws_hbm,
                fdir_hbm, pex_hbm, dots_hbm,
                pack_v, wneg_v, xf_v, xown_v, r_v, p_v, acc_v, dw_v,
                rhs_v, zbuf_v, dread_v, dwrite_v, aw_v):
    wid = lax.axis_index("s")
    base = wid * NPT
    NG = NPT // _L
    EG = EPT // _L
    NCH = EPT // CH
    CG_ = CH // _L
    APT = NPT + _L
    zeros = jnp.zeros((_L,), jnp.float32)
    nown = jnp.minimum(N - base, NPT)
    iota = lax.iota(jnp.int32, _L)
    plane_off = iota * APT

    pltpu.sync_copy(pack_hbm.at[wid], pack_v)
    pltpu.sync_copy(wneg_hbm.at[wid], wneg_v)
    xf_v[pl.ds(XPAD - _L, _L)] = zeros

    def aw_zero():
        def zb(i, carry):
            aw_v[pl.ds(i * _L, _L)] = zeros
            return carry

        lax.fori_loop(0, APT, zb, 0)

    def aw_reduce_into(dst_ref):
        def rb(i, carry):
            sl = pl.ds(i * _L, _L)
            tot = aw_v[sl]
            for l in range(1, _L):
                tot = tot + aw_v[pl.ds(l * APT + i * _L, _L)]
            dst_ref[sl] = dst_ref[sl] + tot
            return carry

        lax.fori_loop(0, NG, rb, 0)

    aw_zero()

    def deg_body(g, carry):
        sl = pl.ds(g * _L, _L)
        pk = pack_v[sl]
        wn = wneg_v[sl]
        rw = (pk >> 14) & 0x3FF
        plsc.addupdate_scatter(aw_v, [rw + plane_off], zeros - wn)
        return carry

    lax.fori_loop(0, EG, deg_body, 0)
    for k in range(NG):
        dw_v[pl.ds(k * _L, _L)] = zeros
    aw_reduce_into(dw_v)

    if node_rhs:
        pltpu.sync_copy(rhs_hbm.at[pl.ds(base, NPT)], rhs_v)
    else:
        aw_zero()

        def chunk_body(c, carry):
            pltpu.sync_copy(rhs_hbm.at[pl.ds(wid * EPT + c * CH, CH)], zbuf_v)

            def g_body(g, carry2):
                pk = pack_v[pl.ds(c * CH + g * _L, _L)]
                zv = zbuf_v[pl.ds(g * _L, _L)]
                rw = (pk >> 14) & 0x3FF
                sgn = (pk >> 24) & 1
                plsc.addupdate_scatter(aw_v, [rw + plane_off],
                                       jnp.where(sgn == 1, -zv, zv))
                return carry2

            lax.fori_loop(0, CG_, g_body, 0)
            return carry

        lax.fori_loop(0, NCH, chunk_body, 0)
        for k in range(NG):
            rhs_v[pl.ds(k * _L, _L)] = zeros
        aw_reduce_into(rhs_v)

    pltpu.sync_copy(pws_hbm.at[pl.ds(base, NPT)], xown_v)
    pltpu.sync_copy(pws_hbm, xf_v.at[pl.ds(0, NPT * _NT)])
    l0 = zeros
    l1 = zeros
    l2 = zeros
    for k in range(NG):
        sl = pl.ds(k * _L, _L)
        rv = rhs_v[sl]
        l0 = l0 + rv
        l1 = l1 + rv * rv
        l2 = l2 + xown_v[sl]
    dwrite_v[pl.ds(0, _L)] = l0
    pltpu.sync_copy(dwrite_v, dots_hbm.at[pl.ds(wid * _L, _L)])
    dwrite_v[pl.ds(0, _L)] = l1
    pltpu.sync_copy(dwrite_v, dots_hbm.at[pl.ds(256 + wid * _L, _L)])
    dwrite_v[pl.ds(0, _L)] = l2
    pltpu.sync_copy(dwrite_v, dots_hbm.at[pl.ds(512 + wid * _L, _L)])
    plsc.subcore_barrier()

    def read_slot(slot):
        pltpu.sync_copy(dots_hbm.at[pl.ds(slot * 256, 256)], dread_v)
        tot = zeros
        for t in range(_NT):
            tot = tot + dread_v[pl.ds(t * _L, _L)]
        return jnp.sum(tot)

    rsum = read_slot(0)
    rsq = read_slot(1)
    psum0 = read_slot(2)
    rmean = rsum * (1.0 / float(N))
    bnorm2 = rsq - float(N) * rmean * rmean
    thresh2 = (tol * tol) * bnorm2

    for k in range(NG):
        sl = pl.ds(k * _L, _L)
        msk = (iota + (k * _L)) < nown
        rhs_v[sl] = rhs_v[sl] - jnp.where(msk, rmean, 0.0)

    def matvec(psum, xs_ref):
        aw_zero()

        def e_body(g, carry):
            sl = pl.ds(g * _L, _L)
            pk = pack_v[sl]
            wn = wneg_v[sl]
            nb = pk & 0x3FFF
            rw = (pk >> 14) & 0x3FF
            xv = plsc.load_gather(xf_v, [nb])
            plsc.addupdate_scatter(aw_v, [rw + plane_off], wn * xv)
            return carry

        lax.fori_loop(0, EG, e_body, 0)
        smean = psum * (1.0 / float(N))
        for k in range(NG):
            sl = pl.ds(k * _L, _L)
            msk = (iota + (k * _L)) < nown
            acc_v[sl] = dw_v[sl] * xs_ref[sl] + jnp.where(msk, smean, 0.0)
        aw_reduce_into(acc_v)

    matvec(psum0, xown_v)
    lr = zeros
    lp = zeros
    for k in range(NG):
        sl = pl.ds(k * _L, _L)
        rv = rhs_v[sl] - acc_v[sl]
        r_v[sl] = rv
        p_v[sl] = rv
        lr = lr + rv * rv
        lp = lp + rv
    dwrite_v[pl.ds(0, _L)] = lr
    pltpu.sync_copy(dwrite_v, dots_hbm.at[pl.ds(wid * _L, _L)])
    dwrite_v[pl.ds(0, _L)] = lp
    pltpu.sync_copy(dwrite_v, dots_hbm.at[pl.ds(256 + wid * _L, _L)])
    pltpu.sync_copy(p_v, pex_hbm.at[pl.ds(base, NPT)])
    plsc.subcore_barrier()
    gamma0 = read_slot(0)
    psum = read_slot(1)
    pltpu.sync_copy(pex_hbm, xf_v.at[pl.ds(0, NPT * _NT)])

    def cond(st):
        k, gamma, _ = st
        return (k < maxiter) & (gamma > thresh2)

    def body(st):
        k, gamma, psum_ = st
        matvec(psum_, p_v)
        lpap = zeros
        for kk in range(NG):
            sl = pl.ds(kk * _L, _L)
            lpap = lpap + p_v[sl] * acc_v[sl]
        dwrite_v[pl.ds(0, _L)] = lpap
        pltpu.sync_copy(dwrite_v, dots_hbm.at[pl.ds(wid * _L, _L)])
        plsc.subcore_barrier()
        pap = read_slot(0)
        alpha_v = (zeros + gamma) / (zeros + pap)
        lrr = zeros
        for kk in range(NG):
            sl = pl.ds(kk * _L, _L)
            xv = xown_v[sl] + alpha_v * p_v[sl]
            rv = r_v[sl] - alpha_v * acc_v[sl]
            xown_v[sl] = xv
            r_v[sl] = rv
            lrr = lrr + rv * rv
        dwrite_v[pl.ds(0, _L)] = lrr
        pltpu.sync_copy(dwrite_v, dots_hbm.at[pl.ds(256 + wid * _L, _L)])
        plsc.subcore_barrier()
        g2 = read_slot(1)
        beta_v = (zeros + g2) / (zeros + gamma)
        lps = zeros
        for kk in range(NG):
            sl = pl.ds(kk * _L, _L)
            pv = r_v[sl] + beta_v * p_v[sl]
            p_v[sl] = pv
            lps = lps + pv
        pltpu.sync_copy(p_v, pex_hbm.at[pl.ds(base, NPT)])
        dwrite_v[pl.ds(0, _L)] = lps
        pltpu.sync_copy(dwrite_v, dots_hbm.at[pl.ds(512 + wid * _L, _L)])
        plsc.subcore_barrier()
        psum_n = read_slot(2)
        pltpu.sync_copy(pex_hbm, xf_v.at[pl.ds(0, NPT * _NT)])
        return k + 1, g2, psum_n

    lax.while_loop(cond, body, (jnp.int32(0), gamma0, psum))

    pltpu.sync_copy(xown_v, pex_hbm.at[pl.ds(base, NPT)])
    plsc.subcore_barrier()
    pltpu.sync_copy(pex_hbm, xf_v.at[pl.ds(0, NPT * _NT)])

    def ep_chunk(c, carry):
        if not node_rhs:
            pltpu.sync_copy(rhs_hbm.at[pl.ds(wid * EPT + c * CH, CH)], zbuf_v)

        def g_body(g, carry2):
            sl = pl.ds(g * _L, _L)
            pk = pack_v[pl.ds(c * CH + g * _L, _L)]
            nb = pk & 0x3FFF
            rw = (pk >> 14) & 0x3FF
            sgn = (pk >> 24) & 1
            pown = plsc.load_gather(xf_v, [rw + base])
            pnbr = plsc.load_gather(xf_v, [nb])
            diff = jnp.where(sgn == 1, pnbr - pown, pown - pnbr)
            if node_rhs:
                val = diff
            else:
                wn = wneg_v[pl.ds(c * CH + g * _L, _L)]
                val = zbuf_v[sl] + wn * diff
            zbuf_v[sl] = val
            return carry2

        lax.fori_loop(0, CG_, g_body, 0)
        pltpu.sync_copy(zbuf_v, fdir_hbm.at[pl.ds(wid * EPT + c * CH, CH)])
        return carry

    lax.fori_loop(0, NCH, ep_chunk, 0)


def _make_solve(N, N_PAD, NPT, EPT, XPAD, CH, tol, maxiter, node_rhs):
    NCH = EPT // CH
    mesh = plsc.VectorSubcoreMesh(core_axis_name="c", subcore_axis_name="s",
                                  num_cores=1)
    return pl.kernel(
        partial(_solve_body, N, NPT, EPT, XPAD, CH, tol, maxiter, node_rhs),
        out_type=(
            jax.ShapeDtypeStruct((_NT * EPT,), jnp.float32),
            jax.ShapeDtypeStruct((N_PAD,), jnp.float32),
            jax.ShapeDtypeStruct((768,), jnp.float32),
        ),
        mesh=mesh,
        compiler_params=pltpu.CompilerParams(needs_layout_passes=False),
        scratch_types=[
            pltpu.VMEM((EPT,), jnp.int32),
            pltpu.VMEM((EPT,), jnp.float32),
            pltpu.VMEM((XPAD,), jnp.float32),
            pltpu.VMEM((NPT,), jnp.float32),
            pltpu.VMEM((NPT,), jnp.float32),
            pltpu.VMEM((NPT,), jnp.float32),
            pltpu.VMEM((NPT + _L,), jnp.float32),
            pltpu.VMEM((NPT,), jnp.float32),
            pltpu.VMEM((NPT,), jnp.float32),
            pltpu.VMEM((CH,), jnp.float32),
            pltpu.VMEM((256,), jnp.float32),
            pltpu.VMEM((_L,), jnp.float32),
            pltpu.VMEM(((NPT + _L) * _L,), jnp.float32),
        ],
    )


def _z_body(scal_ref, fcy_ref, fcut_ref, w_ref, z_ref):
    fc = fcy_ref[...]
    wv = w_ref[...]
    y = (fc + fcut_ref[...]) / wv
    acc = (1.0 / DMAX) * y
    for k in range(HIDDEN):
        t = y * scal_ref[1, k] + scal_ref[2, k]
        sp = jnp.log1p(jnp.exp(-jnp.abs(t))) + jnp.maximum(t, 0.0)
        acc = acc + scal_ref[0, k] * sp
    z_ref[...] = fc - DMIN * wv * acc


def _make_zmap(rows):
    blk = 8
    for cand in range(8, min(rows, 2048) + 1, 8):
        if rows % cand == 0:
            blk = cand
    grid = rows // blk
    return pl.pallas_call(
        _z_body,
        grid=(grid,),
        in_specs=[
            pl.BlockSpec((8, 128), lambda i: (0, 0)),
            pl.BlockSpec((blk, 128), lambda i: (i, 0)),
            pl.BlockSpec((blk, 128), lambda i: (i, 0)),
            pl.BlockSpec((blk, 128), lambda i: (i, 0)),
        ],
        out_specs=pl.BlockSpec((blk, 128), lambda i: (i, 0)),
        out_shape=jax.ShapeDtypeStruct((rows, 128), jnp.float32),
    )


def _invscat_body(EPT, EPB, SENT, pos_hbm, out_hbm, pos_v, val_v, fill_v):
    wid = lax.axis_index("s")
    CPT = pos_v.shape[0]
    FCH = fill_v.shape[0]
    iota = lax.iota(jnp.int32, _L)
    sent = jnp.zeros((_L,), jnp.int32) + SENT

    def fb(i, carry):
        fill_v[pl.ds(i * _L, _L)] = sent
        return carry

    lax.fori_loop(0, FCH // _L, fb, 0)
    for c in range(EPT // FCH):
        pltpu.sync_copy(fill_v, out_hbm.at[pl.ds(wid * EPT + c * FCH, FCH)])

    @pl.when(wid == 0)
    def _():
        pltpu.sync_copy(fill_v.at[pl.ds(0, 128)],
                        out_hbm.at[pl.ds(_NT * EPT, 128)])

    pltpu.sync_copy(pos_hbm.at[wid], pos_v)
    plsc.subcore_barrier()

    def cb(c, carry):
        base = wid * EPB + c * 128
        for g in range(128 // _L):
            val_v[pl.ds(g * _L, _L)] = base + (g * _L) + iota
        pltpu.sync_copy(val_v, out_hbm.at[pos_v.at[c]])
        return carry

    lax.fori_loop(0, CPT, cb, 0)


def _make_invscat(EPT, TPC, EPB, SENT):
    CPT = TPC // 128
    mesh = plsc.VectorSubcoreMesh(core_axis_name="c", subcore_axis_name="s",
                                  num_cores=1)
    return pl.kernel(
        partial(_invscat_body, EPT, EPB, SENT),
        out_type=jax.ShapeDtypeStruct((_NT * EPT + 128,), jnp.int32),
        mesh=mesh,
        compiler_params=pltpu.CompilerParams(needs_layout_passes=False),
        scratch_types=[
            pltpu.VMEM((CPT, 128), jnp.int32),
            pltpu.VMEM((128,), jnp.int32),
            pltpu.VMEM((EPT // 8, ), jnp.int32),
        ],
    )


def kernel(u, edge_index, edge_weights, alpha, log_s, b, num_nodes):
    N = u.shape[0]
    M = edge_weights.shape[0]
    M2 = 2 * M
    NPT = _L * ((N + _NT * _L - 1) // (_NT * _L))
    N_PAD = _NT * NPT
    XPAD = N_PAD + _L
    EPT = 128 * ((M2 // _NT + 4200 + 127) // 128)
    NROW = EPT // 128
    CH = EPT // 10 if EPT % 10 == 0 and (EPT // 10) % _L == 0 else EPT // 8
    while EPT % CH or CH % _L:
        CH -= _L
    NCH = EPT // CH

    src = edge_index[0].astype(jnp.int32)
    dst = edge_index[1].astype(jnp.int32)
    w = edge_weights

    nodes = jnp.concatenate([src, dst])
    nbrs = jnp.concatenate([dst, src])
    wd = jnp.concatenate([w, w])
    sg = jnp.concatenate([jnp.zeros((M,), jnp.int32),
                          jnp.ones((M,), jnp.int32)])
    tile = nodes // NPT
    row = nodes - tile * NPT
    K = 512
    M2P = K * ((M2 + K - 1) // K)
    tile_p = jnp.concatenate(
        [tile, jnp.full((M2P - M2,), _NT + 1, jnp.int32)]) if M2P > M2 else tile
    ohf = (tile_p[:, None] == jnp.arange(_NT, dtype=jnp.int32)[None, :]
           ).astype(jnp.float32).reshape(M2P // K, K, _NT)
    bsum = ohf.sum(axis=1)
    basec = jnp.cumsum(bsum, axis=0) - bsum
    tri = jnp.tril(jnp.ones((K, K), jnp.float32))
    within = jnp.einsum("kj,bjt->bkt", tri, ohf,
                        preferred_element_type=jnp.float32)
    rank_all = (basec[:, None, :] + within).reshape(M2P, _NT)[:M2]
    rank = jnp.take_along_axis(
        rank_all, tile[:, None], axis=1)[:, 0].astype(jnp.int32) - 1
    valid = rank < EPT
    pack_vals = nbrs | (row << 14) | (sg << 24)
    DUMMY = _NT * EPT
    arang = jnp.arange(M2, dtype=jnp.int32)
    pos = jnp.where(valid, tile * EPT + rank, DUMMY + (arang % 128))
    EPB = M2 // _NT
    TPC = 128 * ((EPB + 127) // 128)
    pos3 = jnp.full((_NT, TPC - EPB), DUMMY, jnp.int32)
    pos3 = jnp.concatenate([pos.reshape(_NT, EPB), pos3],
                           axis=1).reshape(_NT, TPC // 128, 128)
    invscat = _make_invscat(EPT, TPC, EPB, M2)
    inv = invscat(pos3)[:_NT * EPT]
    filled = inv < M2
    invc = jnp.minimum(inv, M2 - 1)
    lane = jnp.arange(_NT * EPT, dtype=jnp.int32) % _L
    pack = jnp.where(filled, pack_vals[invc],
                     (NPT + lane) << 14).reshape(_NT, EPT)
    wneg = jnp.where(filled, -wd[invc], 0.0).reshape(_NT, EPT)
    uneg = jnp.where(filled, -1.0, 0.0).reshape(_NT, EPT)
    wtc = jnp.where(filled, wd[invc], 1.0).reshape(_NT * EPT // 128, 128)
    plus_pos = jnp.minimum(pos[:M], _NT * EPT - 1)

    solve_a = _make_solve(N, N_PAD, NPT, EPT, XPAD, CH, TOL_A, MAXIT_A, True)
    solve_b = _make_solve(N, N_PAD, NPT, EPT, XPAD, CH, TOL_B, MAXIT_B, False)
    zmap = _make_zmap(_NT * EPT // 128)

    u_pad = jnp.zeros((N_PAD,), jnp.float32).at[:N].set(u)
    zeros_p = jnp.zeros((N_PAD,), jnp.float32)

    fcut_flat, _, _ = solve_a(pack, uneg, u_pad, zeros_p)
    fcut_rows = fcut_flat.reshape(_NT * EPT // 128, 128)

    pvec = jax.nn.softmax(alpha)
    svec = jnp.exp(log_s)
    scal = jnp.zeros((8, 128), jnp.float32)
    scal = scal.at[0, :HIDDEN].set(pvec * (1.0 / DMIN - 1.0 / DMAX) / svec)
    scal = scal.at[1, :HIDDEN].set(svec)
    scal = scal.at[2, :HIDDEN].set(b)

    def body(i, carry):
        f_cyc_rows, p_ws = carry
        z_rows = zmap(scal, f_cyc_rows, fcut_rows, wtc)
        f_new_flat, p_new, _ = solve_b(pack, wneg,
                                       z_rows.reshape(-1), p_ws)
        return f_new_flat.reshape(_NT * EPT // 128, 128), p_new

    f0 = jnp.zeros((_NT * EPT // 128, 128), jnp.float32)
    f_cyc_rows, _ = lax.fori_loop(0, MITR, body, (f0, zeros_p))

    return (f_cyc_rows.reshape(-1) + fcut_flat)[plus_pos]
